# baseline (speedup 1.0000x reference)
import functools

import jax
import jax.numpy as jnp
from jax import lax
from jax.experimental import pallas as pl
from jax.experimental.pallas import tpu as pltpu
from jax.experimental.pallas import tpu_sc as plsc


def _make_sc_kernel(E, N, D, NW, CHUNK):
    e_per_w = E // NW
    n_full = e_per_w // CHUNK
    tail = e_per_w - n_full * CHUNK
    n_pipe = n_full if n_full % 2 == 0 else n_full - 1
    mesh = plsc.VectorSubcoreMesh(core_axis_name="c", subcore_axis_name="s")

    @functools.partial(
        pl.kernel,
        mesh=mesh,
        out_type=jax.ShapeDtypeStruct((E, 3 * D), jnp.float32),
        scratch_types=[
            pltpu.VMEM((e_per_w,), jnp.int32),
            pltpu.VMEM((e_per_w,), jnp.int32),
        ] + [pltpu.VMEM((CHUNK, D), jnp.float32)] * 6 + [
            pltpu.SemaphoreType.DMA,
            pltpu.SemaphoreType.DMA,
            pltpu.SemaphoreType.DMA,
            pltpu.SemaphoreType.DMA,
        ],
    )
    def sc_gather(edge_hbm, node_hbm, src_hbm, dst_hbm, out_hbm,
                  srcv, dstv, ev0, sv0, dv0, ev1, sv1, dv1,
                  ls0, ls1, ws0, ws1):
        wid = lax.axis_index("s") * 2 + lax.axis_index("c")
        base = wid * e_per_w
        pltpu.sync_copy(src_hbm.at[pl.ds(base, e_per_w)], srcv)
        pltpu.sync_copy(dst_hbm.at[pl.ds(base, e_per_w)], dstv)

        bufs = ((ev0, sv0, dv0, ls0, ws0), (ev1, sv1, dv1, ls1, ws1))

        def load_copies(c, b, size=CHUNK):
            ev, sv, dv, ls, _ = bufs[b]
            off = c * CHUNK
            row = base + off
            sl = pl.ds(0, size)
            return (
                pltpu.make_async_copy(
                    edge_hbm.at[pl.ds(row, size)], ev.at[sl], ls),
                pltpu.make_async_copy(
                    node_hbm.at[srcv.at[pl.ds(off, size)]], sv.at[sl], ls),
                pltpu.make_async_copy(
                    node_hbm.at[dstv.at[pl.ds(off, size)]], dv.at[sl], ls),
            )

        def write_copies(c, b, size=CHUNK):
            ev, sv, dv, _, ws = bufs[b]
            row = base + c * CHUNK
            sl = pl.ds(0, size)
            return (
                pltpu.make_async_copy(
                    ev.at[sl], out_hbm.at[pl.ds(row, size), pl.ds(0, D)], ws),
                pltpu.make_async_copy(
                    sv.at[sl], out_hbm.at[pl.ds(row, size), pl.ds(D, D)], ws),
                pltpu.make_async_copy(
                    dv.at[sl], out_hbm.at[pl.ds(row, size), pl.ds(2 * D, D)],
                    ws),
            )

        def start(copies):
            for cp in copies:
                cp.start()

        def wait(copies):
            for cp in copies:
                cp.wait()

        start(load_copies(0, 0))
        start(load_copies(1, 1))

        def pair(p, carry):
            c0 = 2 * p
            c1 = c0 + 1
            wait(load_copies(c0, 0))
            start(write_copies(c0, 0))
            wait(load_copies(c1, 1))
            start(write_copies(c1, 1))
            wait(write_copies(c0, 0))

            @pl.when(c0 + 2 < n_pipe)
            def _():
                start(load_copies(c0 + 2, 0))

            wait(write_copies(c1, 1))

            @pl.when(c1 + 2 < n_pipe)
            def _():
                start(load_copies(c1 + 2, 1))

            return carry

        lax.fori_loop(0, n_pipe // 2, pair, 0)

        for c, size in ([(n_pipe, CHUNK)] if n_pipe < n_full else []) + \
                       ([(n_full, tail)] if tail else []):
            start(load_copies(c, 0, size))
            wait(load_copies(c, 0, size))
            start(write_copies(c, 0, size))
            wait(write_copies(c, 0, size))

    return sc_gather


def kernel(edge_feat, node_feat, edge_index):
    E, D = edge_feat.shape
    N = node_feat.shape[0]
    src = edge_index[0].astype(jnp.int32)
    dst = edge_index[1].astype(jnp.int32)
    fn = _make_sc_kernel(E, N, D, NW=32, CHUNK=128)
    return fn(edge_feat, node_feat, src, dst)


# speedup vs baseline: 1.0237x; 1.0237x over previous
import functools

import jax
import jax.numpy as jnp
from jax import lax
from jax.experimental import pallas as pl
from jax.experimental.pallas import tpu as pltpu
from jax.experimental.pallas import tpu_sc as plsc


def _make_sc_kernel(E, N, D, NW, CHUNK):
    e_per_w = E // NW
    n_full = e_per_w // CHUNK
    tail = e_per_w - n_full * CHUNK
    n_pipe = n_full if n_full % 2 == 0 else n_full - 1
    mesh = plsc.VectorSubcoreMesh(core_axis_name="c", subcore_axis_name="s")

    @functools.partial(
        pl.kernel,
        mesh=mesh,
        out_type=jax.ShapeDtypeStruct((E, 3 * D), jnp.float32),
        scratch_types=[
            pltpu.VMEM((e_per_w,), jnp.int32),
            pltpu.VMEM((e_per_w,), jnp.int32),
            pltpu.VMEM((CHUNK, 3 * D), jnp.float32),
            pltpu.VMEM((CHUNK, 3 * D), jnp.float32),
            pltpu.SemaphoreType.DMA,
            pltpu.SemaphoreType.DMA,
            pltpu.SemaphoreType.DMA,
            pltpu.SemaphoreType.DMA,
        ],
    )
    def sc_gather(edge_hbm, node_hbm, src_hbm, dst_hbm, out_hbm,
                  srcv, dstv, cb0, cb1, ls0, ls1, ws0, ws1):
        wid = lax.axis_index("s") * 2 + lax.axis_index("c")
        base = wid * e_per_w
        pltpu.sync_copy(src_hbm.at[pl.ds(base, e_per_w)], srcv)
        pltpu.sync_copy(dst_hbm.at[pl.ds(base, e_per_w)], dstv)

        bufs = ((cb0, ls0, ws0), (cb1, ls1, ws1))

        def load_copies(c, b, size=CHUNK):
            cb, ls, _ = bufs[b]
            off = c * CHUNK
            row = base + off
            rs = pl.ds(0, size)
            return (
                pltpu.make_async_copy(
                    edge_hbm.at[pl.ds(row, size)],
                    cb.at[rs, pl.ds(0, D)], ls),
                pltpu.make_async_copy(
                    node_hbm.at[srcv.at[pl.ds(off, size)]],
                    cb.at[rs, pl.ds(D, D)], ls),
                pltpu.make_async_copy(
                    node_hbm.at[dstv.at[pl.ds(off, size)]],
                    cb.at[rs, pl.ds(2 * D, D)], ls),
            )

        def write_copies(c, b, size=CHUNK):
            cb, _, ws = bufs[b]
            row = base + c * CHUNK
            return (
                pltpu.make_async_copy(
                    cb.at[pl.ds(0, size)], out_hbm.at[pl.ds(row, size)], ws),
            )

        def start(copies):
            for cp in copies:
                cp.start()

        def wait(copies):
            for cp in copies:
                cp.wait()

        start(load_copies(0, 0))
        start(load_copies(1, 1))

        def pair(p, carry):
            c0 = 2 * p
            c1 = c0 + 1
            wait(load_copies(c0, 0))
            start(write_copies(c0, 0))
            wait(load_copies(c1, 1))
            start(write_copies(c1, 1))
            wait(write_copies(c0, 0))

            @pl.when(c0 + 2 < n_pipe)
            def _():
                start(load_copies(c0 + 2, 0))

            wait(write_copies(c1, 1))

            @pl.when(c1 + 2 < n_pipe)
            def _():
                start(load_copies(c1 + 2, 1))

            return carry

        lax.fori_loop(0, n_pipe // 2, pair, 0)

        for c, size in ([(n_pipe, CHUNK)] if n_pipe < n_full else []) + \
                       ([(n_full, tail)] if tail else []):
            start(load_copies(c, 0, size))
            wait(load_copies(c, 0, size))
            start(write_copies(c, 0, size))
            wait(write_copies(c, 0, size))

    return sc_gather


def kernel(edge_feat, node_feat, edge_index):
    E, D = edge_feat.shape
    N = node_feat.shape[0]
    src = edge_index[0].astype(jnp.int32)
    dst = edge_index[1].astype(jnp.int32)
    fn = _make_sc_kernel(E, N, D, NW=32, CHUNK=80)
    return fn(edge_feat, node_feat, src, dst)


# speedup vs baseline: 1.0258x; 1.0020x over previous
import functools

import jax
import jax.numpy as jnp
from jax import lax
from jax.experimental import pallas as pl
from jax.experimental.pallas import tpu as pltpu
from jax.experimental.pallas import tpu_sc as plsc


def _make_sc_kernel(E, N, D, NW, CHUNK, NBUF=4, SKEW=2):
    e_per_w = E // NW
    n_full = e_per_w // CHUNK
    tail = e_per_w - n_full * CHUNK
    n_pipe = n_full // NBUF * NBUF
    mesh = plsc.VectorSubcoreMesh(core_axis_name="c", subcore_axis_name="s")

    @functools.partial(
        pl.kernel,
        mesh=mesh,
        out_type=jax.ShapeDtypeStruct((E, 3 * D), jnp.float32),
        scratch_types=[
            pltpu.VMEM((e_per_w,), jnp.int32),
            pltpu.VMEM((e_per_w,), jnp.int32),
        ] + [pltpu.VMEM((CHUNK, 3 * D), jnp.float32)] * NBUF
          + [pltpu.SemaphoreType.DMA] * (2 * NBUF),
    )
    def sc_gather(edge_hbm, node_hbm, src_hbm, dst_hbm, out_hbm,
                  srcv, dstv, *scratch):
        cbs = scratch[:NBUF]
        lsems = scratch[NBUF:2 * NBUF]
        wsems = scratch[2 * NBUF:]
        wid = lax.axis_index("s") * 2 + lax.axis_index("c")
        base = wid * e_per_w
        pltpu.sync_copy(src_hbm.at[pl.ds(base, e_per_w)], srcv)
        pltpu.sync_copy(dst_hbm.at[pl.ds(base, e_per_w)], dstv)

        bufs = tuple(
            (cbs[b], lsems[b], wsems[b]) for b in range(NBUF))

        def load_copies(c, b, size=CHUNK):
            cb, ls, _ = bufs[b]
            off = c * CHUNK
            row = base + off
            rs = pl.ds(0, size)
            return (
                pltpu.make_async_copy(
                    edge_hbm.at[pl.ds(row, size)],
                    cb.at[rs, pl.ds(0, D)], ls),
                pltpu.make_async_copy(
                    node_hbm.at[srcv.at[pl.ds(off, size)]],
                    cb.at[rs, pl.ds(D, D)], ls),
                pltpu.make_async_copy(
                    node_hbm.at[dstv.at[pl.ds(off, size)]],
                    cb.at[rs, pl.ds(2 * D, D)], ls),
            )

        def write_copies(c, b, size=CHUNK):
            cb, _, ws = bufs[b]
            row = base + c * CHUNK
            return (
                pltpu.make_async_copy(
                    cb.at[pl.ds(0, size)], out_hbm.at[pl.ds(row, size)], ws),
            )

        def start(copies):
            for cp in copies:
                cp.start()

        def wait(copies):
            for cp in copies:
                cp.wait()

        for c in range(SKEW):
            start(load_copies(c, c % NBUF))

        def group(g, carry):
            for b in range(NBUF):
                c = g * NBUF + b
                cl = c + SKEW
                bl = (b + SKEW) % NBUF

                @pl.when((cl >= NBUF) & (cl < n_pipe))
                def _():
                    wait(write_copies(cl - NBUF, bl))

                @pl.when(cl < n_pipe)
                def _():
                    start(load_copies(cl, bl))

                wait(load_copies(c, b))
                start(write_copies(c, b))
            return carry

        lax.fori_loop(0, n_pipe // NBUF, group, 0)
        for j in range(NBUF):
            c = n_pipe - NBUF + j
            wait(write_copies(c, c % NBUF))

        for c, size in ([(n_pipe, CHUNK)] if n_pipe < n_full else []) + \
                       ([(n_full, tail)] if tail else []):
            start(load_copies(c, 0, size))
            wait(load_copies(c, 0, size))
            start(write_copies(c, 0, size))
            wait(write_copies(c, 0, size))

    return sc_gather


def kernel(edge_feat, node_feat, edge_index):
    E, D = edge_feat.shape
    N = node_feat.shape[0]
    src = edge_index[0].astype(jnp.int32)
    dst = edge_index[1].astype(jnp.int32)
    fn = _make_sc_kernel(E, N, D, NW=32, CHUNK=64)
    return fn(edge_feat, node_feat, src, dst)
